# baseline (speedup 1.0000x reference)
import functools

import jax
import jax.numpy as jnp
from jax import lax
from jax.experimental import pallas as pl
from jax.experimental.pallas import tpu as pltpu
from jax.experimental.pallas import tpu_sc as plsc

_NC = 2
_NS = 16
_NW = _NC * _NS
_L = 16

_N = 10000
_NPAD = 10240
_E = 320000
_EPAD = 327680
_D = 128
_ROWS_PER_TILE = _NPAD // _NS
_EPW = _EPAD // _NW
_IDXROWS_PW = _EPW // 128
_EPT_DEG = _EPAD // _NS
_IDXROWS_DEG = _EPT_DEG // 128

_mesh = plsc.VectorSubcoreMesh(core_axis_name="c", subcore_axis_name="s")


def _rsqrt16(d):
    t = d
    scale = jnp.full((_L,), 1.0, jnp.float32)
    for _ in range(10):
        c = t >= 4.0
        t = jnp.where(c, t * 0.25, t)
        scale = jnp.where(c, scale * 0.5, scale)
    y = 7.0 / 6.0 - t * (1.0 / 6.0)
    for _ in range(5):
        y = y * (1.5 - 0.5 * t * y * y)
    return y * scale


@functools.partial(
    pl.kernel,
    out_type=(
        jax.ShapeDtypeStruct((_NPAD,), jnp.float32),
        jax.ShapeDtypeStruct((_EPAD,), jnp.float32),
    ),
    mesh=_mesh,
    compiler_params=pltpu.CompilerParams(needs_layout_passes=False),
    scratch_types=(
        pltpu.VMEM_SHARED((_NPAD,), jnp.float32),
        pltpu.VMEM((16, 128), jnp.int32),
        pltpu.VMEM((16, 128), jnp.int32),
        pltpu.VMEM((2048,), jnp.float32),
        pltpu.VMEM((2048,), jnp.float32),
        pltpu.VMEM((_ROWS_PER_TILE,), jnp.float32),
        pltpu.VMEM((_ROWS_PER_TILE,), jnp.float32),
        pltpu.VMEM((_NPAD,), jnp.float32),
    ),
)
def _pre_kernel(row2d_hbm, col2d_hbm, ew_hbm, selfnorm_hbm, norm_hbm,
                deg_sh, col_v, row_v, ew_v, norm_v, slice_v, self_v, dinv_v):
    cid = lax.axis_index("c")
    sid = lax.axis_index("s")
    wid = cid * _NS + sid

    zero16 = jnp.zeros((_L,), jnp.float32)

    @pl.loop(0, _ROWS_PER_TILE // _L)
    def _(i):
        slice_v[pl.ds(i * _L, _L)] = zero16

    pltpu.sync_copy(slice_v, deg_sh.at[pl.ds(sid * _ROWS_PER_TILE, _ROWS_PER_TILE)])
    plsc.subcore_barrier()

    base_row = sid * _IDXROWS_DEG

    @pl.loop(0, _IDXROWS_DEG // 16)
    def _(ch):
        r0 = base_row + ch * 16
        pltpu.sync_copy(col2d_hbm.at[pl.ds(r0, 16)], col_v)
        pltpu.sync_copy(ew_hbm.at[pl.ds(r0 * 128, 2048)], ew_v)
        for j in range(16):
            pltpu.sync_copy(ew_v.at[pl.ds(j * 128, 128)],
                            deg_sh.at[col_v.at[j]], add=True)
    plsc.subcore_barrier()

    off = sid * _ROWS_PER_TILE
    pltpu.sync_copy(deg_sh.at[pl.ds(off, _ROWS_PER_TILE)], slice_v)

    @pl.loop(0, _ROWS_PER_TILE // _L)
    def _(i):
        d = slice_v[pl.ds(i * _L, _L)] + 1.0
        y = _rsqrt16(d)
        slice_v[pl.ds(i * _L, _L)] = y
        self_v[pl.ds(i * _L, _L)] = y * y

    pltpu.sync_copy(slice_v, deg_sh.at[pl.ds(off, _ROWS_PER_TILE)])

    @pl.when(cid == 0)
    def _():
        pltpu.sync_copy(self_v, selfnorm_hbm.at[pl.ds(off, _ROWS_PER_TILE)])

    plsc.subcore_barrier()

    pltpu.sync_copy(deg_sh, dinv_v)

    @pl.loop(0, _IDXROWS_PW // 16)
    def _(ch):
        r0 = wid * _IDXROWS_PW + ch * 16
        pltpu.sync_copy(row2d_hbm.at[pl.ds(r0, 16)], row_v)
        pltpu.sync_copy(col2d_hbm.at[pl.ds(r0, 16)], col_v)
        pltpu.sync_copy(ew_hbm.at[pl.ds(r0 * 128, 2048)], ew_v)

        @pl.loop(0, 16)
        def _(kk):
            for j in range(8):
                r16 = row_v[kk, pl.ds(j * _L, _L)]
                c16 = col_v[kk, pl.ds(j * _L, _L)]
                w16 = ew_v[pl.ds(kk * 128 + j * _L, _L)]
                a = plsc.load_gather(dinv_v, [r16])
                b = plsc.load_gather(dinv_v, [c16])
                norm_v[pl.ds(kk * 128 + j * _L, _L)] = a * w16 * b

        pltpu.sync_copy(norm_v, norm_hbm.at[pl.ds(r0 * 128, 2048)])


_CBL = 128
_NCHL = _EPW // _CBL
_GS = 4
_GR = _CBL // _GS
_NCH0 = 36
_NCH1 = (_EPAD // 128 - _NS * _NCH0) // _NS


@functools.partial(
    pl.kernel,
    out_type=jax.ShapeDtypeStruct((_NC, _NPAD, _D), jnp.float32),
    mesh=_mesh,
    compiler_params=pltpu.CompilerParams(needs_layout_passes=False),
    scratch_types=(
        pltpu.VMEM_SHARED((_NPAD, _D), jnp.float32),
        pltpu.VMEM((_CBL, _D), jnp.float32),
        pltpu.VMEM((_CBL, _D), jnp.float32),
        pltpu.VMEM((1, 128), jnp.int32),
        pltpu.VMEM((1, 128), jnp.int32),
        pltpu.VMEM((1, 128), jnp.int32),
        pltpu.VMEM((1, 128), jnp.int32),
        pltpu.VMEM((_CBL,), jnp.float32),
        pltpu.VMEM((_CBL,), jnp.float32),
        pltpu.SemaphoreType.DMA,
        pltpu.SemaphoreType.DMA,
        pltpu.SemaphoreType.DMA,
        pltpu.SemaphoreType.DMA,
        pltpu.SemaphoreType.DMA,
        pltpu.SemaphoreType.DMA,
        pltpu.SemaphoreType.DMA,
        pltpu.SemaphoreType.DMA,
    ),
)
def _layer_kernel(h_hbm, row2d_hbm, col2d_hbm, norm_hbm, out_hbm,
                  acc_sh, rows0, rows1, idx0, idx1, ccol0, ccol1,
                  norm0, norm1, sem0, sem1, semi0, semi1, sems0, sems1,
                  semc0, semc1):
    cid = lax.axis_index("c")
    sid = lax.axis_index("s")
    wid = cid * _NS + sid
    nch = jnp.where(cid == 0, _NCH0, _NCH1)
    base = jnp.where(cid == 0, sid * _NCH0, _NS * _NCH0 + sid * _NCH1)

    zero16 = jnp.zeros((_L,), jnp.float32)

    @pl.loop(0, _CBL)
    def _(i):
        for j in range(8):
            rows0[i, pl.ds(j * _L, _L)] = zero16

    off = sid * _ROWS_PER_TILE
    for k in range(_ROWS_PER_TILE // _CBL):
        pltpu.sync_copy(rows0, acc_sh.at[pl.ds(off + k * _CBL, _CBL)])
    plsc.subcore_barrier()

    pltpu.sync_copy(row2d_hbm.at[pl.ds(base, 1)], idx0)
    pltpu.sync_copy(col2d_hbm.at[pl.ds(base, 1)], ccol0)
    pltpu.sync_copy(norm_hbm.at[pl.ds(base * 128, _CBL)], norm0)
    pltpu.async_copy(row2d_hbm.at[pl.ds(base + 1, 1)], idx1, semi1)
    pltpu.async_copy(norm_hbm.at[pl.ds((base + 1) * 128, _CBL)], norm1, semi1)
    for t in range(_GS):
        pltpu.async_copy(h_hbm.at[idx0.at[0, pl.ds(t * _GR, _GR)]],
                         rows0.at[pl.ds(t * _GR, _GR)], sem0)

    bufs = ((rows0, idx0, ccol0, norm0, sem0, semi0, sems0, semc0),
            (rows1, idx1, ccol1, norm1, sem1, semi1, sems1, semc1))

    def _body(ch, b, first):
        rows_c, idx_c, col_c, norm_c, sem_c, semi_c, sems_c, semc_c = bufs[b]
        rows_n, idx_n, col_n, norm_n, sem_n, semi_n, sems_n, semc_n = (
            bufs[1 - b])

        pltpu.make_async_copy(row2d_hbm.at[pl.ds(base, 1)], idx_n,
                              semi_n).wait()
        pltpu.make_async_copy(norm_hbm.at[pl.ds(base * 128, _CBL)],
                              norm_n, semi_n).wait()
        if not first:
            pltpu.make_async_copy(rows_n, acc_sh.at[col_n.at[0]],
                                  sems_n).wait()
        nxt1 = ch + 1
        nxt1 = jnp.where(nxt1 >= nch, nxt1 - nch, nxt1)
        pltpu.async_copy(col2d_hbm.at[pl.ds(base + nxt1, 1)], col_n, semc_n)
        for t in range(_GS):
            pltpu.async_copy(h_hbm.at[idx_n.at[0, pl.ds(t * _GR, _GR)]],
                             rows_n.at[pl.ds(t * _GR, _GR)], sem_n)

        pltpu.make_async_copy(h_hbm.at[idx_c.at[0]], rows_c, sem_c).wait()

        @pl.loop(0, _CBL // _L)
        def _(g):
            n16 = norm_c[pl.ds(g * _L, _L)]
            for k in range(_L):
                e = g * _L + k
                s = n16[k]
                for j in range(8):
                    rows_c[e, pl.ds(j * _L, _L)] = (
                        rows_c[e, pl.ds(j * _L, _L)] * s)

        if not first:
            pltpu.make_async_copy(col2d_hbm.at[pl.ds(base, 1)], col_c,
                                  semc_c).wait()
        pltpu.async_copy(rows_c, acc_sh.at[col_c.at[0]], sems_c, add=True)

        nxt = ch + 2
        nxt = jnp.where(nxt >= nch, nxt - nch, nxt)
        pltpu.async_copy(row2d_hbm.at[pl.ds(base + nxt, 1)], idx_c, semi_c)
        pltpu.async_copy(norm_hbm.at[pl.ds((base + nxt) * 128, _CBL)],
                         norm_c, semi_c)

    _body(jnp.int32(0), 0, True)
    _body(jnp.int32(1), 1, False)

    @pl.loop(2, nch, step=2)
    def _(i):
        for b in range(2):
            _body(i + b, b, False)

    pltpu.make_async_copy(rows1, acc_sh.at[ccol1.at[0]], sems1).wait()
    pltpu.make_async_copy(row2d_hbm.at[pl.ds(base, 1)], idx1, semi1).wait()
    pltpu.make_async_copy(norm_hbm.at[pl.ds(base * 128, _CBL)], norm1,
                          semi1).wait()
    pltpu.make_async_copy(col2d_hbm.at[pl.ds(base, 1)], ccol0, semc0).wait()
    pltpu.make_async_copy(h_hbm.at[idx0.at[0]], rows0, sem0).wait()

    plsc.subcore_barrier()

    for k in range(_ROWS_PER_TILE // 128):
        pltpu.sync_copy(acc_sh.at[pl.ds(off + k * 128, 128)],
                        out_hbm.at[cid, pl.ds(off + k * 128, 128)])


_BLK = 1000


def _mm1_body(x_ref, w_ref, o_ref):
    o_ref[...] = jnp.dot(x_ref[...], w_ref[...],
                         preferred_element_type=jnp.float32)


def _tc2_body(pa_ref, pb_ref, h_ref, sn_ref, b_ref, w_ref, o_ref):
    pre = pa_ref[...] + pb_ref[...] + h_ref[...] * sn_ref[...] + b_ref[...]
    z = jnp.maximum(pre, 0.0)
    o_ref[...] = jnp.dot(z, w_ref[...], preferred_element_type=jnp.float32)


def _tc3_body(pa_ref, pb_ref, h_ref, sn_ref, b_ref, batch_ref, w3_ref, b3_ref,
              o_ref, s_acc, c_acc):
    i = pl.program_id(0)

    @pl.when(i == 0)
    def _():
        s_acc[...] = jnp.zeros_like(s_acc)
        c_acc[...] = jnp.zeros_like(c_acc)

    pre = pa_ref[...] + pb_ref[...] + h_ref[...] * sn_ref[...] + b_ref[...]
    z = jnp.maximum(pre, 0.0)
    gids = jax.lax.broadcasted_iota(jnp.int32, (_BLK, 64), 1)
    mask = jnp.where(batch_ref[...] == gids, 1.0, 0.0)
    s_acc[...] += lax.dot_general(mask, z, (((0,), (0,)), ((), ())),
                                  preferred_element_type=jnp.float32)
    c_acc[...] += lax.dot_general(mask, jnp.ones_like(z),
                                  (((0,), (0,)), ((), ())),
                                  preferred_element_type=jnp.float32)

    @pl.when(i == pl.num_programs(0) - 1)
    def _():
        pooled = s_acc[...] / jnp.maximum(c_acc[...], 1.0)
        o_ref[...] = jnp.dot(pooled, w3_ref[...],
                             preferred_element_type=jnp.float32) + b3_ref[...]


def kernel(x, edge_index, edge_weight, batch, W1, b1, W2, b2, W3, b3):
    row = edge_index[0]
    col = edge_index[1]
    pad = _EPAD - _E
    rowp = jnp.concatenate([row, jnp.zeros((pad,), jnp.int32)])
    colp = jnp.concatenate([col, jnp.full((pad,), _NPAD - 1, jnp.int32)])
    ewp = jnp.concatenate([edge_weight, jnp.zeros((pad,), jnp.float32)])
    row2d = rowp.reshape(-1, 128)
    col2d = colp.reshape(-1, 128)

    selfnorm, norm = _pre_kernel(row2d, col2d, ewp)
    sn2d = selfnorm[:_N].reshape(-1, 1)

    nblk = _N // _BLK
    row_spec = pl.BlockSpec((_BLK, _D), lambda i: (i, 0))
    pad_spec = pl.BlockSpec((_BLK, _D), lambda i: (i, 0))
    sn_spec = pl.BlockSpec((_BLK, 1), lambda i: (i, 0))
    full_spec = pl.BlockSpec((_D, _D), lambda i: (0, 0))
    bias_spec = pl.BlockSpec((1, _D), lambda i: (0, 0))

    h1 = pl.pallas_call(
        _mm1_body,
        grid=(nblk,),
        in_specs=[row_spec, full_spec],
        out_specs=row_spec,
        out_shape=jax.ShapeDtypeStruct((_N, _D), jnp.float32),
    )(x, W1)

    part1 = _layer_kernel(h1, row2d, col2d, norm)

    h2 = pl.pallas_call(
        _tc2_body,
        grid=(nblk,),
        in_specs=[pad_spec, pad_spec, row_spec, sn_spec, bias_spec, full_spec],
        out_specs=row_spec,
        out_shape=jax.ShapeDtypeStruct((_N, _D), jnp.float32),
    )(part1[0], part1[1], h1, sn2d, b1.reshape(1, -1), W2)

    part2 = _layer_kernel(h2, row2d, col2d, norm)

    W3p = jnp.pad(W3, ((0, 0), (0, _D - W3.shape[1])))
    b3p = jnp.pad(b3, (0, _D - b3.shape[0])).reshape(1, -1)
    batch2d = batch.reshape(-1, 1)

    out128 = pl.pallas_call(
        _tc3_body,
        grid=(nblk,),
        in_specs=[pad_spec, pad_spec, row_spec, sn_spec, bias_spec,
                  pl.BlockSpec((_BLK, 1), lambda i: (i, 0)),
                  full_spec, bias_spec],
        out_specs=pl.BlockSpec((64, _D), lambda i: (0, 0)),
        out_shape=jax.ShapeDtypeStruct((64, _D), jnp.float32),
        scratch_shapes=[pltpu.VMEM((64, _D), jnp.float32),
                        pltpu.VMEM((64, _D), jnp.float32)],
    )(part2[0], part2[1], h2, sn2d, b2.reshape(1, -1), batch2d, W3p, b3p)

    return out128[:, :W3.shape[1]]


# speedup vs baseline: 1.1506x; 1.1506x over previous
import functools

import jax
import jax.numpy as jnp
from jax import lax
from jax.experimental import pallas as pl
from jax.experimental.pallas import tpu as pltpu
from jax.experimental.pallas import tpu_sc as plsc

_NC = 2
_NS = 16
_NW = _NC * _NS
_L = 16

_N = 10000
_NPAD = 10240
_E = 320000
_EPAD = 327680
_D = 128
_ROWS_PER_TILE = _NPAD // _NS
_EPW = _EPAD // _NW
_IDXROWS_PW = _EPW // 128
_EPT_DEG = _EPAD // _NS
_IDXROWS_DEG = _EPT_DEG // 128

_mesh = plsc.VectorSubcoreMesh(core_axis_name="c", subcore_axis_name="s")


def _rsqrt16(d):
    t = d
    scale = jnp.full((_L,), 1.0, jnp.float32)
    for _ in range(10):
        c = t >= 4.0
        t = jnp.where(c, t * 0.25, t)
        scale = jnp.where(c, scale * 0.5, scale)
    y = 7.0 / 6.0 - t * (1.0 / 6.0)
    for _ in range(5):
        y = y * (1.5 - 0.5 * t * y * y)
    return y * scale


@functools.partial(
    pl.kernel,
    out_type=(
        jax.ShapeDtypeStruct((_NPAD,), jnp.float32),
        jax.ShapeDtypeStruct((_EPAD,), jnp.float32),
    ),
    mesh=_mesh,
    compiler_params=pltpu.CompilerParams(needs_layout_passes=False),
    scratch_types=(
        pltpu.VMEM_SHARED((_NPAD,), jnp.float32),
        pltpu.VMEM((16, 128), jnp.int32),
        pltpu.VMEM((16, 128), jnp.int32),
        pltpu.VMEM((2048,), jnp.float32),
        pltpu.VMEM((2048,), jnp.float32),
        pltpu.VMEM((_ROWS_PER_TILE,), jnp.float32),
        pltpu.VMEM((_ROWS_PER_TILE,), jnp.float32),
        pltpu.VMEM((_NPAD,), jnp.float32),
    ),
)
def _pre_kernel(row2d_hbm, col2d_hbm, ew_hbm, selfnorm_hbm, norm_hbm,
                deg_sh, col_v, row_v, ew_v, norm_v, slice_v, self_v, dinv_v):
    cid = lax.axis_index("c")
    sid = lax.axis_index("s")
    wid = cid * _NS + sid

    zero16 = jnp.zeros((_L,), jnp.float32)

    @pl.loop(0, _ROWS_PER_TILE // _L)
    def _(i):
        slice_v[pl.ds(i * _L, _L)] = zero16

    pltpu.sync_copy(slice_v, deg_sh.at[pl.ds(sid * _ROWS_PER_TILE, _ROWS_PER_TILE)])
    plsc.subcore_barrier()

    base_row = sid * _IDXROWS_DEG

    @pl.loop(0, _IDXROWS_DEG // 16)
    def _(ch):
        r0 = base_row + ch * 16
        pltpu.sync_copy(col2d_hbm.at[pl.ds(r0, 16)], col_v)
        pltpu.sync_copy(ew_hbm.at[pl.ds(r0 * 128, 2048)], ew_v)
        for j in range(16):
            pltpu.sync_copy(ew_v.at[pl.ds(j * 128, 128)],
                            deg_sh.at[col_v.at[j]], add=True)
    plsc.subcore_barrier()

    off = sid * _ROWS_PER_TILE
    pltpu.sync_copy(deg_sh.at[pl.ds(off, _ROWS_PER_TILE)], slice_v)

    @pl.loop(0, _ROWS_PER_TILE // _L)
    def _(i):
        d = slice_v[pl.ds(i * _L, _L)] + 1.0
        y = _rsqrt16(d)
        slice_v[pl.ds(i * _L, _L)] = y
        self_v[pl.ds(i * _L, _L)] = y * y

    pltpu.sync_copy(slice_v, deg_sh.at[pl.ds(off, _ROWS_PER_TILE)])

    @pl.when(cid == 0)
    def _():
        pltpu.sync_copy(self_v, selfnorm_hbm.at[pl.ds(off, _ROWS_PER_TILE)])

    plsc.subcore_barrier()

    pltpu.sync_copy(deg_sh, dinv_v)

    @pl.loop(0, _IDXROWS_PW // 16)
    def _(ch):
        r0 = wid * _IDXROWS_PW + ch * 16
        pltpu.sync_copy(row2d_hbm.at[pl.ds(r0, 16)], row_v)
        pltpu.sync_copy(col2d_hbm.at[pl.ds(r0, 16)], col_v)
        pltpu.sync_copy(ew_hbm.at[pl.ds(r0 * 128, 2048)], ew_v)

        @pl.loop(0, 16)
        def _(kk):
            for j in range(8):
                r16 = row_v[kk, pl.ds(j * _L, _L)]
                c16 = col_v[kk, pl.ds(j * _L, _L)]
                w16 = ew_v[pl.ds(kk * 128 + j * _L, _L)]
                a = plsc.load_gather(dinv_v, [r16])
                b = plsc.load_gather(dinv_v, [c16])
                norm_v[pl.ds(kk * 128 + j * _L, _L)] = a * w16 * b

        pltpu.sync_copy(norm_v, norm_hbm.at[pl.ds(r0 * 128, 2048)])


_CBL = 128
_NCHL = _EPW // _CBL
_GS = 4
_GR = _CBL // _GS
_NCH0 = 124
_NCH1 = (_EPAD // 128 - _NS * _NCH0) // _NS


@functools.partial(
    pl.kernel,
    out_type=jax.ShapeDtypeStruct((_NC, _NPAD, _D), jnp.float32),
    mesh=_mesh,
    compiler_params=pltpu.CompilerParams(needs_layout_passes=False),
    scratch_types=(
        pltpu.VMEM_SHARED((_NPAD, _D), jnp.float32),
        pltpu.VMEM((_CBL, _D), jnp.float32),
        pltpu.VMEM((_CBL, _D), jnp.float32),
        pltpu.VMEM((1, 128), jnp.int32),
        pltpu.VMEM((1, 128), jnp.int32),
        pltpu.VMEM((1, 128), jnp.int32),
        pltpu.VMEM((1, 128), jnp.int32),
        pltpu.VMEM((_CBL,), jnp.float32),
        pltpu.VMEM((_CBL,), jnp.float32),
        pltpu.SemaphoreType.DMA,
        pltpu.SemaphoreType.DMA,
        pltpu.SemaphoreType.DMA,
        pltpu.SemaphoreType.DMA,
        pltpu.SemaphoreType.DMA,
        pltpu.SemaphoreType.DMA,
        pltpu.SemaphoreType.DMA,
        pltpu.SemaphoreType.DMA,
    ),
)
def _layer_kernel(h_hbm, row2d_hbm, col2d_hbm, norm_hbm, out_hbm,
                  acc_sh, rows0, rows1, idx0, idx1, ccol0, ccol1,
                  norm0, norm1, sem0, sem1, semi0, semi1, sems0, sems1,
                  semc0, semc1):
    cid = lax.axis_index("c")
    sid = lax.axis_index("s")
    wid = cid * _NS + sid
    nch = jnp.where(cid == 0, _NCH0, _NCH1)
    base = jnp.where(cid == 0, sid * _NCH0, _NS * _NCH0 + sid * _NCH1)

    zero16 = jnp.zeros((_L,), jnp.float32)

    @pl.loop(0, _CBL)
    def _(i):
        for j in range(8):
            rows0[i, pl.ds(j * _L, _L)] = zero16

    off = sid * _ROWS_PER_TILE
    for k in range(_ROWS_PER_TILE // _CBL):
        pltpu.sync_copy(rows0, acc_sh.at[pl.ds(off + k * _CBL, _CBL)])
    plsc.subcore_barrier()

    pltpu.sync_copy(row2d_hbm.at[pl.ds(base, 1)], idx0)
    pltpu.sync_copy(col2d_hbm.at[pl.ds(base, 1)], ccol0)
    pltpu.sync_copy(norm_hbm.at[pl.ds(base * 128, _CBL)], norm0)
    pltpu.async_copy(row2d_hbm.at[pl.ds(base + 1, 1)], idx1, semi1)
    pltpu.async_copy(norm_hbm.at[pl.ds((base + 1) * 128, _CBL)], norm1, semi1)
    for t in range(_GS):
        pltpu.async_copy(h_hbm.at[idx0.at[0, pl.ds(t * _GR, _GR)]],
                         rows0.at[pl.ds(t * _GR, _GR)], sem0)

    bufs = ((rows0, idx0, ccol0, norm0, sem0, semi0, sems0, semc0),
            (rows1, idx1, ccol1, norm1, sem1, semi1, sems1, semc1))

    def _body(ch, b, first):
        rows_c, idx_c, col_c, norm_c, sem_c, semi_c, sems_c, semc_c = bufs[b]
        rows_n, idx_n, col_n, norm_n, sem_n, semi_n, sems_n, semc_n = (
            bufs[1 - b])

        pltpu.make_async_copy(row2d_hbm.at[pl.ds(base, 1)], idx_n,
                              semi_n).wait()
        pltpu.make_async_copy(norm_hbm.at[pl.ds(base * 128, _CBL)],
                              norm_n, semi_n).wait()
        if not first:
            pltpu.make_async_copy(rows_n, acc_sh.at[col_n.at[0]],
                                  sems_n).wait()
        nxt1 = ch + 1
        nxt1 = jnp.where(nxt1 >= nch, nxt1 - nch, nxt1)
        pltpu.async_copy(col2d_hbm.at[pl.ds(base + nxt1, 1)], col_n, semc_n)
        for t in range(_GS):
            pltpu.async_copy(h_hbm.at[idx_n.at[0, pl.ds(t * _GR, _GR)]],
                             rows_n.at[pl.ds(t * _GR, _GR)], sem_n)

        pltpu.make_async_copy(h_hbm.at[idx_c.at[0]], rows_c, sem_c).wait()

        @pl.loop(0, _CBL // _L)
        def _(g):
            n16 = norm_c[pl.ds(g * _L, _L)]
            for k in range(_L):
                e = g * _L + k
                s = n16[k]
                for j in range(8):
                    rows_c[e, pl.ds(j * _L, _L)] = (
                        rows_c[e, pl.ds(j * _L, _L)] * s)

        if not first:
            pltpu.make_async_copy(col2d_hbm.at[pl.ds(base, 1)], col_c,
                                  semc_c).wait()
        pltpu.async_copy(rows_c, acc_sh.at[col_c.at[0]], sems_c, add=True)

        nxt = ch + 2
        nxt = jnp.where(nxt >= nch, nxt - nch, nxt)
        pltpu.async_copy(row2d_hbm.at[pl.ds(base + nxt, 1)], idx_c, semi_c)
        pltpu.async_copy(norm_hbm.at[pl.ds((base + nxt) * 128, _CBL)],
                         norm_c, semi_c)

    _body(jnp.int32(0), 0, True)
    _body(jnp.int32(1), 1, False)

    @pl.loop(2, nch, step=2)
    def _(i):
        for b in range(2):
            _body(i + b, b, False)

    pltpu.make_async_copy(rows1, acc_sh.at[ccol1.at[0]], sems1).wait()
    pltpu.make_async_copy(row2d_hbm.at[pl.ds(base, 1)], idx1, semi1).wait()
    pltpu.make_async_copy(norm_hbm.at[pl.ds(base * 128, _CBL)], norm1,
                          semi1).wait()
    pltpu.make_async_copy(col2d_hbm.at[pl.ds(base, 1)], ccol0, semc0).wait()
    pltpu.make_async_copy(h_hbm.at[idx0.at[0]], rows0, sem0).wait()

    plsc.subcore_barrier()

    for k in range(_ROWS_PER_TILE // 128):
        pltpu.sync_copy(acc_sh.at[pl.ds(off + k * 128, 128)],
                        out_hbm.at[cid, pl.ds(off + k * 128, 128)])


_BLK = 1000


def _mm1_body(x_ref, w_ref, o_ref):
    o_ref[...] = jnp.dot(x_ref[...], w_ref[...],
                         preferred_element_type=jnp.float32)


def _tc2_body(pa_ref, pb_ref, h_ref, sn_ref, b_ref, w_ref, o_ref):
    pre = pa_ref[...] + pb_ref[...] + h_ref[...] * sn_ref[...] + b_ref[...]
    z = jnp.maximum(pre, 0.0)
    o_ref[...] = jnp.dot(z, w_ref[...], preferred_element_type=jnp.float32)


def _tc3_body(pa_ref, pb_ref, h_ref, sn_ref, b_ref, batch_ref, w3_ref, b3_ref,
              o_ref, s_acc, c_acc):
    i = pl.program_id(0)

    @pl.when(i == 0)
    def _():
        s_acc[...] = jnp.zeros_like(s_acc)
        c_acc[...] = jnp.zeros_like(c_acc)

    pre = pa_ref[...] + pb_ref[...] + h_ref[...] * sn_ref[...] + b_ref[...]
    z = jnp.maximum(pre, 0.0)
    gids = jax.lax.broadcasted_iota(jnp.int32, (_BLK, 64), 1)
    mask = jnp.where(batch_ref[...] == gids, 1.0, 0.0)
    s_acc[...] += lax.dot_general(mask, z, (((0,), (0,)), ((), ())),
                                  preferred_element_type=jnp.float32)
    c_acc[...] += lax.dot_general(mask, jnp.ones_like(z),
                                  (((0,), (0,)), ((), ())),
                                  preferred_element_type=jnp.float32)

    @pl.when(i == pl.num_programs(0) - 1)
    def _():
        pooled = s_acc[...] / jnp.maximum(c_acc[...], 1.0)
        o_ref[...] = jnp.dot(pooled, w3_ref[...],
                             preferred_element_type=jnp.float32) + b3_ref[...]


def kernel(x, edge_index, edge_weight, batch, W1, b1, W2, b2, W3, b3):
    row = edge_index[0]
    col = edge_index[1]
    pad = _EPAD - _E
    rowp = jnp.concatenate([row, jnp.zeros((pad,), jnp.int32)])
    colp = jnp.concatenate([col, jnp.full((pad,), _NPAD - 1, jnp.int32)])
    ewp = jnp.concatenate([edge_weight, jnp.zeros((pad,), jnp.float32)])
    row2d = rowp.reshape(-1, 128)
    col2d = colp.reshape(-1, 128)

    selfnorm, norm = _pre_kernel(row2d, col2d, ewp)
    sn2d = selfnorm[:_N].reshape(-1, 1)

    nblk = _N // _BLK
    row_spec = pl.BlockSpec((_BLK, _D), lambda i: (i, 0))
    pad_spec = pl.BlockSpec((_BLK, _D), lambda i: (i, 0))
    sn_spec = pl.BlockSpec((_BLK, 1), lambda i: (i, 0))
    full_spec = pl.BlockSpec((_D, _D), lambda i: (0, 0))
    bias_spec = pl.BlockSpec((1, _D), lambda i: (0, 0))

    h1 = pl.pallas_call(
        _mm1_body,
        grid=(nblk,),
        in_specs=[row_spec, full_spec],
        out_specs=row_spec,
        out_shape=jax.ShapeDtypeStruct((_N, _D), jnp.float32),
    )(x, W1)

    part1 = _layer_kernel(h1, row2d, col2d, norm)

    h2 = pl.pallas_call(
        _tc2_body,
        grid=(nblk,),
        in_specs=[pad_spec, pad_spec, row_spec, sn_spec, bias_spec, full_spec],
        out_specs=row_spec,
        out_shape=jax.ShapeDtypeStruct((_N, _D), jnp.float32),
    )(part1[0], part1[1], h1, sn2d, b1.reshape(1, -1), W2)

    part2 = _layer_kernel(h2, row2d, col2d, norm)

    W3p = jnp.pad(W3, ((0, 0), (0, _D - W3.shape[1])))
    b3p = jnp.pad(b3, (0, _D - b3.shape[0])).reshape(1, -1)
    batch2d = batch.reshape(-1, 1)

    out128 = pl.pallas_call(
        _tc3_body,
        grid=(nblk,),
        in_specs=[pad_spec, pad_spec, row_spec, sn_spec, bias_spec,
                  pl.BlockSpec((_BLK, 1), lambda i: (i, 0)),
                  full_spec, bias_spec],
        out_specs=pl.BlockSpec((64, _D), lambda i: (0, 0)),
        out_shape=jax.ShapeDtypeStruct((64, _D), jnp.float32),
        scratch_shapes=[pltpu.VMEM((64, _D), jnp.float32),
                        pltpu.VMEM((64, _D), jnp.float32)],
    )(part2[0], part2[1], h2, sn2d, b2.reshape(1, -1), batch2d, W3p, b3p)

    return out128[:, :W3.shape[1]]
